# baseline (speedup 1.0000x reference)
import functools

import jax
import jax.numpy as jnp
from jax import lax
from jax.experimental import pallas as pl
from jax.experimental.pallas import tpu as pltpu
from jax.experimental.pallas import tpu_sc as plsc

DIM = 4096
NUM_EXPERTS = 64
HEAD = 4
H = NUM_EXPERTS * HEAD
TOP_K = 8
TOKENS = 4 * 2048
BT = 1024

NW = 32
NCHUNK = 2
CTOK = TOKENS // NCHUNK
TPW = CTOK // NW


def _tc_body(x_ref, wg_ref, bg_ref, wp_ref, bp_ref, scores_ref):
    h = jnp.dot(x_ref[...], wg_ref[...], preferred_element_type=jnp.float32)
    h = h + bg_ref[...]
    h = 0.5 * h * (1.0 + jax.lax.erf(h * 0.7071067811865476))
    logits = jnp.dot(h, wp_ref[...], preferred_element_type=jnp.float32)
    logits = logits + bp_ref[...]
    m = jnp.max(logits, axis=-1, keepdims=True)
    e = jnp.exp(logits - m)
    scores_ref[...] = e / jnp.sum(e, axis=-1, keepdims=True)


def _merge_top16(a, b):
    h = jnp.maximum(a, lax.rev(b, dimensions=(0,)))
    return plsc.sort_key_val(h, h, descending=True)[0]


def _sc_mask_body(scores_hbm, out_hbm, s_v, o_v):
    wid = lax.axis_index("s") * 2 + lax.axis_index("c")
    base = wid * TPW
    pltpu.sync_copy(scores_hbm.at[pl.ds(base, TPW)], s_v)

    lane = lax.iota(jnp.int32, 16)
    sel7 = lane == TOP_K - 1
    one = jnp.ones((16,), jnp.int32)
    zero = jnp.zeros((16,), jnp.int32)

    @plsc.parallel_loop(0, TPW, 1, unroll=2)
    def _token(t):
        v = [s_v[t, pl.ds(16 * j, 16)] for j in range(4)]
        s = [plsc.sort_key_val(vj, vj, descending=True)[0] for vj in v]
        h = _merge_top16(_merge_top16(s[0], s[1]), _merge_top16(s[2], s[3]))
        t8 = jnp.full((16,), lax.reduce_max(jnp.where(sel7, h, -jnp.inf), axes=(0,)))
        gt = [vj > t8 for vj in v]
        cnt = (plsc.all_reduce_population_count(gt[0])
               + plsc.all_reduce_population_count(gt[1])
               + plsc.all_reduce_population_count(gt[2])
               + plsc.all_reduce_population_count(gt[3]))
        need = TOP_K - cnt
        tot = zero
        for j in range(4):
            eq = v[j] == t8
            pc = plsc.cumsum(jnp.where(eq, one, zero))
            take = eq & ((tot + pc) <= need)
            o_v[t, pl.ds(16 * j, 16)] = jnp.where(gt[j] | take, 1.0, 0.0)
            tot = tot + plsc.all_reduce_population_count(eq)

    pltpu.sync_copy(o_v, out_hbm.at[pl.ds(base, TPW)])


_sc_mask = pl.kernel(
    _sc_mask_body,
    out_type=jax.ShapeDtypeStruct((CTOK, NUM_EXPERTS), jnp.float32),
    mesh=plsc.VectorSubcoreMesh(core_axis_name="c", subcore_axis_name="s"),
    scratch_types=[
        pltpu.VMEM((TPW, NUM_EXPERTS), jnp.float32),
        pltpu.VMEM((TPW, NUM_EXPERTS), jnp.float32),
    ],
    compiler_params=pltpu.CompilerParams(needs_layout_passes=False),
)


def _tc_chunk(xc, W_gate, bg2d, W_proj, bp2d):
    grid = CTOK // BT
    return pl.pallas_call(
        _tc_body,
        grid=(grid,),
        in_specs=[
            pl.BlockSpec((BT, DIM), lambda i: (i, 0)),
            pl.BlockSpec((DIM, H), lambda i: (0, 0)),
            pl.BlockSpec((1, H), lambda i: (0, 0)),
            pl.BlockSpec((H, NUM_EXPERTS), lambda i: (0, 0)),
            pl.BlockSpec((1, NUM_EXPERTS), lambda i: (0, 0)),
        ],
        out_specs=pl.BlockSpec((BT, NUM_EXPERTS), lambda i: (i, 0)),
        out_shape=jax.ShapeDtypeStruct((CTOK, NUM_EXPERTS), jnp.float32),
    )(xc, W_gate, bg2d, W_proj, bp2d)


@jax.jit
def _gate(x2d, W_gate, b_gate, W_proj, b_proj):
    bg2d = b_gate.reshape(1, H)
    bp2d = b_proj.reshape(1, NUM_EXPERTS)
    scores_c = []
    routed_c = []
    for c in range(NCHUNK):
        sc = _tc_chunk(
            jax.lax.slice_in_dim(x2d, c * CTOK, (c + 1) * CTOK, axis=0),
            W_gate, bg2d, W_proj, bp2d,
        )
        scores_c.append(sc)
        routed_c.append(_sc_mask(sc))
    return jnp.concatenate(scores_c, axis=0), jnp.concatenate(routed_c, axis=0)


def kernel(x, W_gate, b_gate, W_proj, b_proj):
    B, T, _ = x.shape
    scores, routed = _gate(x.reshape(B * T, DIM), W_gate, b_gate, W_proj, b_proj)
    gate_scores = scores.reshape(B, T, NUM_EXPERTS)
    routed_experts = routed.reshape(B, T, NUM_EXPERTS)
    return (gate_scores, routed_experts, jnp.float32(0.0))


# speedup vs baseline: 2.9188x; 2.9188x over previous
import functools

import jax
import jax.numpy as jnp
from jax import lax
from jax.experimental import pallas as pl
from jax.experimental.pallas import tpu as pltpu
from jax.experimental.pallas import tpu_sc as plsc

DIM = 4096
NUM_EXPERTS = 64
HEAD = 4
H = NUM_EXPERTS * HEAD
TOP_K = 8
TOKENS = 4 * 2048
BT = 1024

NW = 32
NCHUNK = 1
CTOK = TOKENS // NCHUNK
TPW = CTOK // NW


def _tc_body(x_ref, wg_ref, bg_ref, wp_ref, bp_ref, scores_ref, routed_ref):
    h = jnp.dot(x_ref[...], wg_ref[...], preferred_element_type=jnp.float32)
    h = h + bg_ref[...]
    h = 0.5 * h * (1.0 + jax.lax.erf(h * 0.7071067811865476))
    logits = jnp.dot(h, wp_ref[...], preferred_element_type=jnp.float32)
    logits = logits + bp_ref[...]

    lt = logits.T
    m = jnp.max(lt, axis=0, keepdims=True)
    e = jnp.exp(lt - m)
    sT = e / jnp.sum(e, axis=0, keepdims=True)
    scores_ref[...] = sT.T

    work = sT
    cum = jnp.zeros((1, BT), jnp.float32)
    t8 = jnp.full((1, BT), -1.0, jnp.float32)
    crossed = jnp.zeros((1, BT), jnp.bool_)
    for _ in range(TOP_K):
        cur = jnp.max(work, axis=0, keepdims=True)
        eq = work == cur
        cnt = jnp.sum(jnp.where(eq, 1.0, 0.0), axis=0, keepdims=True)
        newcum = cum + cnt
        now = jnp.logical_and(newcum >= float(TOP_K), jnp.logical_not(crossed))
        t8 = jnp.where(now, cur, t8)
        crossed = jnp.logical_or(crossed, now)
        work = jnp.where(eq, -1.0, work)
        cum = newcum

    gt = sT > t8
    eqm = sT == t8
    eqf = jnp.where(eqm, 1.0, 0.0)
    r_idx = jax.lax.broadcasted_iota(jnp.int32, (NUM_EXPERTS, NUM_EXPERTS), 0)
    c_idx = jax.lax.broadcasted_iota(jnp.int32, (NUM_EXPERTS, NUM_EXPERTS), 1)
    ltri = jnp.where(c_idx <= r_idx, 1.0, 0.0).astype(jnp.bfloat16)
    prefix = jnp.dot(ltri, eqf.astype(jnp.bfloat16),
                     preferred_element_type=jnp.float32)
    cntgt = jnp.sum(jnp.where(gt, 1.0, 0.0), axis=0, keepdims=True)
    need = float(TOP_K) - cntgt
    mask = jnp.logical_or(gt, jnp.logical_and(eqm, prefix <= need))
    routed_ref[...] = jnp.where(mask, 1.0, 0.0).T


def _merge_top16(a, b):
    h = jnp.maximum(a, lax.rev(b, dimensions=(0,)))
    return plsc.sort_key_val(h, h, descending=True)[0]


def _sc_mask_body(scores_hbm, out_hbm, s_v, o_v):
    wid = lax.axis_index("s") * 2 + lax.axis_index("c")
    base = wid * TPW
    pltpu.sync_copy(scores_hbm.at[pl.ds(base, TPW)], s_v)

    lane = lax.iota(jnp.int32, 16)
    sel7 = lane == TOP_K - 1
    one = jnp.ones((16,), jnp.int32)
    zero = jnp.zeros((16,), jnp.int32)

    @plsc.parallel_loop(0, TPW, 1, unroll=2)
    def _token(t):
        v = [s_v[t, pl.ds(16 * j, 16)] for j in range(4)]
        s = [plsc.sort_key_val(vj, vj, descending=True)[0] for vj in v]
        h = _merge_top16(_merge_top16(s[0], s[1]), _merge_top16(s[2], s[3]))
        t8 = jnp.full((16,), lax.reduce_max(jnp.where(sel7, h, -jnp.inf), axes=(0,)))
        gt = [vj > t8 for vj in v]
        cnt = (plsc.all_reduce_population_count(gt[0])
               + plsc.all_reduce_population_count(gt[1])
               + plsc.all_reduce_population_count(gt[2])
               + plsc.all_reduce_population_count(gt[3]))
        need = TOP_K - cnt
        tot = zero
        for j in range(4):
            eq = v[j] == t8
            pc = plsc.cumsum(jnp.where(eq, one, zero))
            take = eq & ((tot + pc) <= need)
            o_v[t, pl.ds(16 * j, 16)] = jnp.where(gt[j] | take, 1.0, 0.0)
            tot = tot + plsc.all_reduce_population_count(eq)

    pltpu.sync_copy(o_v, out_hbm.at[pl.ds(base, TPW)])


@functools.cache
def _sc_mask():
    return pl.kernel(
        _sc_mask_body,
        out_type=jax.ShapeDtypeStruct((CTOK, NUM_EXPERTS), jnp.float32),
        mesh=plsc.VectorSubcoreMesh(core_axis_name="c", subcore_axis_name="s"),
        scratch_types=[
            pltpu.VMEM((TPW, NUM_EXPERTS), jnp.float32),
            pltpu.VMEM((TPW, NUM_EXPERTS), jnp.float32),
        ],
        compiler_params=pltpu.CompilerParams(needs_layout_passes=False),
    )


def _tc_chunk(xc, W_gate, bg2d, W_proj, bp2d):
    grid = CTOK // BT
    return pl.pallas_call(
        _tc_body,
        grid=(grid,),
        in_specs=[
            pl.BlockSpec((BT, DIM), lambda i: (i, 0)),
            pl.BlockSpec((DIM, H), lambda i: (0, 0)),
            pl.BlockSpec((1, H), lambda i: (0, 0)),
            pl.BlockSpec((H, NUM_EXPERTS), lambda i: (0, 0)),
            pl.BlockSpec((1, NUM_EXPERTS), lambda i: (0, 0)),
        ],
        out_specs=[
            pl.BlockSpec((BT, NUM_EXPERTS), lambda i: (i, 0)),
            pl.BlockSpec((BT, NUM_EXPERTS), lambda i: (i, 0)),
        ],
        out_shape=[
            jax.ShapeDtypeStruct((CTOK, NUM_EXPERTS), jnp.float32),
            jax.ShapeDtypeStruct((CTOK, NUM_EXPERTS), jnp.float32),
        ],
    )(xc, W_gate, bg2d, W_proj, bp2d)


@jax.jit
def _gate(x2d, W_gate, b_gate, W_proj, b_proj):
    bg2d = b_gate.reshape(1, H)
    bp2d = b_proj.reshape(1, NUM_EXPERTS)
    return _tc_chunk(x2d, W_gate, bg2d, W_proj, bp2d)


def kernel(x, W_gate, b_gate, W_proj, b_proj):
    B, T, _ = x.shape
    scores, routed = _gate(x.reshape(B * T, DIM), W_gate, b_gate, W_proj, b_proj)
    gate_scores = scores.reshape(B, T, NUM_EXPERTS)
    routed_experts = routed.reshape(B, T, NUM_EXPERTS)
    return (gate_scores, routed_experts, jnp.float32(0.0))
